# baseline (speedup 1.0000x reference)
import functools

import jax
import jax.numpy as jnp
from jax import lax
from jax.experimental import pallas as pl
from jax.experimental.pallas import tpu as pltpu
from jax.experimental.pallas import tpu_sc as plsc

BATCH = 16384
N_FIELDS = 26
FIELD_VOCAB = 100000
EMB = 16
IN_DIM = (N_FIELDS + 2) * EMB

NC = 2
NS = 16
NW = NC * NS
ROWS_PER_W = BATCH // NW
CHUNK = 128
NCHUNK = ROWS_PER_W // CHUNK


def _gather_body(user_ids, item_ids, sf_flat, user_table, item_table,
                 sparse_flat, out, idx_v, rows_v, sem):
    wid = lax.axis_index("s") * NC + lax.axis_index("c")
    base = wid * ROWS_PER_W

    def chunk_step(c, carry):
        rb = base + c * CHUNK

        pltpu.sync_copy(user_ids.at[pl.ds(rb, CHUNK)], idx_v)
        pltpu.async_copy(user_table.at[idx_v], rows_v, sem).wait()
        pltpu.sync_copy(rows_v, out.at[0, pl.ds(rb, CHUNK), :])

        pltpu.sync_copy(item_ids.at[pl.ds(rb, CHUNK)], idx_v)
        pltpu.async_copy(item_table.at[idx_v], rows_v, sem).wait()
        pltpu.sync_copy(rows_v, out.at[1, pl.ds(rb, CHUNK), :])

        def field_step(f, carry2):
            pltpu.sync_copy(sf_flat.at[f, pl.ds(rb, CHUNK)], idx_v)
            pltpu.async_copy(sparse_flat.at[idx_v], rows_v, sem).wait()
            pltpu.sync_copy(rows_v, out.at[f + 2, pl.ds(rb, CHUNK), :])
            return carry2

        lax.fori_loop(0, N_FIELDS, field_step, 0)
        return carry

    lax.fori_loop(0, NCHUNK, chunk_step, 0)


_sc_gather = functools.partial(
    pl.kernel,
    mesh=plsc.VectorSubcoreMesh(core_axis_name="c", subcore_axis_name="s"),
    compiler_params=pltpu.CompilerParams(use_tc_tiling_on_sc=False),
    out_type=jax.ShapeDtypeStruct((N_FIELDS + 2, BATCH, EMB), jnp.float32),
    scratch_types=[
        pltpu.VMEM((CHUNK,), jnp.int32),
        pltpu.VMEM((CHUNK, EMB), jnp.float32),
        pltpu.SemaphoreType.DMA,
    ],
)(_gather_body)


PACK = 8
PROWS = BATCH // PACK
BLK_P = 256
NFLD = N_FIELDS + 2


def _dense_body(g_ref, w1bd_ref, w2bd_ref, w3bd_ref, w4bd_ref, ones_ref,
                b1t_ref, b2t_ref, b3t_ref, b4_ref, o_ref):
    f32 = jnp.float32
    acc = jnp.zeros((BLK_P, 128), f32)
    ssum = jnp.zeros((BLK_P, 128), f32)
    sqsum = jnp.zeros((BLK_P, 128), f32)
    for f in range(NFLD):
        x = g_ref[f]
        acc = acc + jnp.dot(x, w1bd_ref[f], preferred_element_type=f32)
        ssum = ssum + x
        sqsum = sqsum + x * x
    h = jnp.maximum(acc + b1t_ref[...], 0.0)
    h = jnp.maximum(jnp.dot(h, w2bd_ref[...], preferred_element_type=f32)
                    + b2t_ref[...], 0.0)
    h = jnp.maximum(jnp.dot(h, w3bd_ref[...], preferred_element_type=f32)
                    + b3t_ref[...], 0.0)
    deep = jnp.dot(h, w4bd_ref[...], preferred_element_type=f32) + b4_ref[0, 0]
    s = jnp.dot(ssum, ones_ref[...], preferred_element_type=f32)
    sq = jnp.dot(sqsum, ones_ref[...], preferred_element_type=f32)
    fm = 0.5 * (s * s - sq)
    o_ref[...] = jax.nn.sigmoid(deep + fm)


def _dense(gp, w1bd, w2bd, w3bd, w4bd, ones_bd, b1t, b2t, b3t, b4):
    wspec2 = lambda shape: pl.BlockSpec(shape, lambda i: (0, 0))
    return pl.pallas_call(
        _dense_body,
        grid=(PROWS // BLK_P,),
        in_specs=[
            pl.BlockSpec((NFLD, BLK_P, 128), lambda i: (0, i, 0)),
            pl.BlockSpec((NFLD, 128, 128), lambda i: (0, 0, 0)),
            wspec2((128, 128)),
            wspec2((128, 128)),
            wspec2((128, PACK)),
            wspec2((128, PACK)),
            wspec2((1, 128)),
            wspec2((1, 128)),
            wspec2((1, 128)),
            wspec2((1, 1)),
        ],
        out_specs=pl.BlockSpec((BLK_P, PACK), lambda i: (i, 0)),
        out_shape=jax.ShapeDtypeStruct((PROWS, PACK), jnp.float32),
    )(gp, w1bd, w2bd, w3bd, w4bd, ones_bd, b1t, b2t, b3t, b4)


def kernel(user_ids, item_ids, sparse_features, user_table, item_table,
           sparse_tables, W1, b1, W2, b2, W3, b3, W4, b4):
    sf_flat = (sparse_features.astype(jnp.int32)
               + jnp.arange(N_FIELDS, dtype=jnp.int32)[None, :] * FIELD_VOCAB).T
    sparse_flat = sparse_tables.reshape(N_FIELDS * FIELD_VOCAB, EMB)
    g = _sc_gather(user_ids, item_ids, sf_flat, user_table, item_table,
                   sparse_flat)
    gp = g.reshape(NFLD, PROWS, 128)

    eye8 = jnp.eye(PACK, dtype=jnp.float32)
    w1bd = jnp.einsum("uv,fdj->fudvj", eye8, W1.reshape(NFLD, EMB, EMB)
                      ).reshape(NFLD, 128, 128)
    w2bd = jnp.kron(eye8, W2)
    w3bd = jnp.kron(eye8, W3)
    w4bd = jnp.kron(eye8, W4)
    ones_bd = jnp.kron(eye8, jnp.ones((EMB, 1), jnp.float32))
    b1t = jnp.tile(b1, PACK).reshape(1, 128)
    b2t = jnp.tile(b2, PACK).reshape(1, 128)
    b3t = jnp.tile(b3, PACK).reshape(1, 128)

    out = _dense(gp, w1bd, w2bd, w3bd, w4bd, ones_bd, b1t, b2t, b3t,
                 b4.reshape(1, 1))
    return out.reshape(BATCH)


# speedup vs baseline: 1.2077x; 1.2077x over previous
import functools

import jax
import jax.numpy as jnp
from jax import lax
from jax.experimental import pallas as pl
from jax.experimental.pallas import tpu as pltpu
from jax.experimental.pallas import tpu_sc as plsc

BATCH = 16384
N_FIELDS = 26
FIELD_VOCAB = 100000
EMB = 16
PACK = 8
NFLD = N_FIELDS + 2
PROWS = BATCH // PACK

NC = 2
NS = 16
NW = NC * NS
ROWS_PER_W = BATCH // NW
CHUNK = 128
NCHUNK = ROWS_PER_W // CHUNK
OUT_PR = ROWS_PER_W // PACK


def _extract_chunk(pack_c, c, w16, outbuf):
    def s_body(s, carry):
        ivec = 16 * s + lax.iota(jnp.int32, 16)
        w16s = w16[pl.ds(pl.multiple_of(128 * c + 16 * s, 16), 16)]
        rowidx = 16 * c + (ivec >> 3)
        lanebase = (ivec & 7) << 4
        for j in range(EMB):
            val = plsc.load_gather(pack_c, [ivec, w16s + j])
            plsc.store_scatter(outbuf, [rowidx, lanebase + j], val)
        return carry

    lax.fori_loop(0, CHUNK // 16, s_body, 0)


def _gather_body(uid, iid, sfidx, up, ip, sp, g_p, idxraw, idxp, w16,
                 pack, outA, outB, g0, g1, g2, g3, wA, wB):
    wid = lax.axis_index("s") * NC + lax.axis_index("c")
    base = pl.multiple_of(wid * ROWS_PER_W, ROWS_PER_W)
    rbase = pl.multiple_of(wid * OUT_PR, OUT_PR)
    gsems = [g0, g1, g2, g3]

    def do_field(idx_hbm, idx_off, table, out_row, outbuf, wsem, first):
        pltpu.sync_copy(idx_hbm.at[pl.ds(idx_off, ROWS_PER_W)], idxraw)
        for s in range(ROWS_PER_W // 16):
            v = idxraw[pl.ds(16 * s, 16)]
            idxp[pl.ds(16 * s, 16)] = v >> 3
            w16[pl.ds(16 * s, 16)] = (v & 7) << 4
        handles = []
        for c in range(NCHUNK):
            handles.append(
                pltpu.async_copy(
                    table.at[idxp.at[pl.ds(128 * c, CHUNK)]],
                    pack.at[c], gsems[c]))
        if not first:
            pltpu.make_async_copy(
                outbuf, g_p.at[0, pl.ds(rbase, OUT_PR), :], wsem).wait()
        for c in range(NCHUNK):
            handles[c].wait()
            _extract_chunk(pack.at[c], c, w16, outbuf)
        pltpu.async_copy(outbuf, g_p.at[out_row, pl.ds(rbase, OUT_PR), :],
                         wsem)

    do_field(uid, base, up, 0, outA, wA, True)
    do_field(iid, base, ip, 1, outB, wB, True)

    def pair_body(m, carry):
        f0 = 2 * m
        do_field(sfidx, pl.multiple_of(f0 * BATCH, BATCH) + base, sp,
                 f0 + 2, outA, wA, False)
        do_field(sfidx, pl.multiple_of((f0 + 1) * BATCH, BATCH) + base, sp,
                 f0 + 3, outB, wB, False)
        return carry

    lax.fori_loop(0, N_FIELDS // 2, pair_body, 0)

    pltpu.make_async_copy(outA, g_p.at[0, pl.ds(rbase, OUT_PR), :], wA).wait()
    pltpu.make_async_copy(outB, g_p.at[0, pl.ds(rbase, OUT_PR), :], wB).wait()


_sc_gather = functools.partial(
    pl.kernel,
    mesh=plsc.VectorSubcoreMesh(core_axis_name="c", subcore_axis_name="s"),
    compiler_params=pltpu.CompilerParams(needs_layout_passes=False),
    out_type=jax.ShapeDtypeStruct((NFLD, PROWS, 128), jnp.float32),
    scratch_types=[
        pltpu.VMEM((ROWS_PER_W,), jnp.int32),
        pltpu.VMEM((ROWS_PER_W,), jnp.int32),
        pltpu.VMEM((ROWS_PER_W,), jnp.int32),
        pltpu.VMEM((NCHUNK, CHUNK, 128), jnp.float32),
        pltpu.VMEM((OUT_PR, 128), jnp.float32),
        pltpu.VMEM((OUT_PR, 128), jnp.float32),
        pltpu.SemaphoreType.DMA,
        pltpu.SemaphoreType.DMA,
        pltpu.SemaphoreType.DMA,
        pltpu.SemaphoreType.DMA,
        pltpu.SemaphoreType.DMA,
        pltpu.SemaphoreType.DMA,
    ],
)(_gather_body)


BLK_P = 256


def _dense_body(g_ref, w1bd_ref, w2bd_ref, w3bd_ref, w4bd_ref, ones_ref,
                b1t_ref, b2t_ref, b3t_ref, b4_ref, o_ref):
    f32 = jnp.float32
    acc = jnp.zeros((BLK_P, 128), f32)
    ssum = jnp.zeros((BLK_P, 128), f32)
    sqsum = jnp.zeros((BLK_P, 128), f32)
    for f in range(NFLD):
        x = g_ref[f]
        acc = acc + jnp.dot(x, w1bd_ref[f], preferred_element_type=f32)
        ssum = ssum + x
        sqsum = sqsum + x * x
    h = jnp.maximum(acc + b1t_ref[...], 0.0)
    h = jnp.maximum(jnp.dot(h, w2bd_ref[...], preferred_element_type=f32)
                    + b2t_ref[...], 0.0)
    h = jnp.maximum(jnp.dot(h, w3bd_ref[...], preferred_element_type=f32)
                    + b3t_ref[...], 0.0)
    deep = jnp.dot(h, w4bd_ref[...], preferred_element_type=f32) + b4_ref[0, 0]
    s = jnp.dot(ssum, ones_ref[...], preferred_element_type=f32)
    sq = jnp.dot(sqsum, ones_ref[...], preferred_element_type=f32)
    fm = 0.5 * (s * s - sq)
    o_ref[...] = jax.nn.sigmoid(deep + fm)


def _dense(gp, w1bd, w2bd, w3bd, w4bd, ones_bd, b1t, b2t, b3t, b4):
    wspec2 = lambda shape: pl.BlockSpec(shape, lambda i: (0, 0))
    return pl.pallas_call(
        _dense_body,
        grid=(PROWS // BLK_P,),
        in_specs=[
            pl.BlockSpec((NFLD, BLK_P, 128), lambda i: (0, i, 0)),
            pl.BlockSpec((NFLD, 128, 128), lambda i: (0, 0, 0)),
            wspec2((128, 128)),
            wspec2((128, 128)),
            wspec2((128, PACK)),
            wspec2((128, PACK)),
            wspec2((1, 128)),
            wspec2((1, 128)),
            wspec2((1, 128)),
            wspec2((1, 1)),
        ],
        out_specs=pl.BlockSpec((BLK_P, PACK), lambda i: (i, 0)),
        out_shape=jax.ShapeDtypeStruct((PROWS, PACK), jnp.float32),
    )(gp, w1bd, w2bd, w3bd, w4bd, ones_bd, b1t, b2t, b3t, b4)


def kernel(user_ids, item_ids, sparse_features, user_table, item_table,
           sparse_tables, W1, b1, W2, b2, W3, b3, W4, b4):
    sfidx = (sparse_features.astype(jnp.int32)
             + jnp.arange(N_FIELDS, dtype=jnp.int32)[None, :] * FIELD_VOCAB
             ).T.reshape(N_FIELDS * BATCH)
    up = user_table.reshape(user_table.shape[0] // PACK, 128)
    ip = item_table.reshape(item_table.shape[0] // PACK, 128)
    sp = sparse_tables.reshape(N_FIELDS * FIELD_VOCAB // PACK, 128)

    gp = _sc_gather(user_ids, item_ids, sfidx, up, ip, sp)

    eye8 = jnp.eye(PACK, dtype=jnp.float32)
    w1bd = jnp.einsum("uv,fdj->fudvj", eye8, W1.reshape(NFLD, EMB, EMB)
                      ).reshape(NFLD, 128, 128)
    w2bd = jnp.kron(eye8, W2)
    w3bd = jnp.kron(eye8, W3)
    w4bd = jnp.kron(eye8, W4)
    ones_bd = jnp.kron(eye8, jnp.ones((EMB, 1), jnp.float32))
    b1t = jnp.tile(b1, PACK).reshape(1, 128)
    b2t = jnp.tile(b2, PACK).reshape(1, 128)
    b3t = jnp.tile(b3, PACK).reshape(1, 128)

    out = _dense(gp, w1bd, w2bd, w3bd, w4bd, ones_bd, b1t, b2t, b3t,
                 b4.reshape(1, 1))
    return out.reshape(BATCH)
